# Initial kernel scaffold; baseline (speedup 1.0000x reference)
import jax
import jax.numpy as jnp
from jax.experimental import pallas as pl


def kernel(charges, d_ij, idx_j, mol_idx):
    raise NotImplementedError("write your pallas kernel here")


# speedup vs baseline: 71.8219x; 71.8219x over previous
import functools

import jax
import jax.numpy as jnp
from jax import lax
from jax.experimental import pallas as pl
from jax.experimental.pallas import tpu as pltpu
from jax.experimental.pallas import tpu_sc as plsc

N = 50000
M = 32
NMOL = 500
RC = 4.6
FACTOR = 13.605693122994 * 0.52917721092

R = 400
NCHUNK = N // R
NW = 32
KMAX = (NCHUNK + NW - 1) // NW
NGROUP = R // 16
ACCP = 512


def _body(chg_hbm, d_hbm, idx_hbm, mol_hbm, out_hbm,
          chg_v, d_v, idx_v, mol_v, acc_v, tmp_v, shared):
    cid = lax.axis_index("c")
    sid = lax.axis_index("s")
    wid = sid * 2 + cid

    pltpu.sync_copy(chg_hbm, chg_v)

    zero16 = jnp.zeros((16,), jnp.float32)

    def zero_body(i, _):
        acc_v[pl.ds(pl.multiple_of(i * 16, 16), 16)] = zero16
        return 0

    lax.fori_loop(0, ACCP // 16, zero_body, 0)

    iota = lax.iota(jnp.int32, 16)
    iota32 = iota * 32
    inv_rc2 = jnp.float32(1.0 / (RC * RC))

    def chunk_work(chunk_id):
        row0 = chunk_id * R
        pltpu.sync_copy(d_hbm.at[pl.ds(pl.multiple_of(row0 * 32, 256), R * 32)], d_v)
        pltpu.sync_copy(idx_hbm.at[pl.ds(pl.multiple_of(row0 * 32, 256), R * 32)], idx_v)
        pltpu.sync_copy(mol_hbm.at[pl.ds(pl.multiple_of(row0, 8), R)], mol_v)

        def group_body(g, _):
            goff = g * (16 * 32)

            def nbody(m, acc16):
                ids = iota32 + (goff + m)
                idxv = plsc.load_gather(idx_v, [ids])
                dv = plsc.load_gather(d_v, [ids])
                qj = plsc.load_gather(chg_v, [idxv])
                x2 = dv * dv * inv_rc2
                inside = x2 < 1.0
                denom = jnp.where(inside, x2 - 1.0, jnp.float32(-1.0))
                fc = jnp.where(inside, 1.0 - jnp.exp(x2 / denom),
                               jnp.float32(1.0))
                return acc16 + fc * qj / dv

            s = lax.fori_loop(0, M, nbody, zero16)

            g16 = g * 16
            qi = chg_v[pl.ds(pl.multiple_of(row0 + g16, 16), 16)]
            e_atom = s * qi * jnp.float32(FACTOR)
            cs = plsc.cumsum(e_atom)
            molv = mol_v[pl.ds(pl.multiple_of(g16, 16), 16)]
            nxt = jnp.minimum(iota + (g16 + 1), R - 1)
            moln = plsc.load_gather(mol_v, [nxt])
            change = molv != moln
            is15 = iota == 15
            endm = change | is15
            boundm = change & jnp.logical_not(is15)
            plsc.addupdate_scatter(acc_v, [molv], cs, mask=endm)
            plsc.addupdate_scatter(acc_v, [moln], -cs, mask=boundm)
            return 0

        lax.fori_loop(0, NGROUP, group_body, 0)

    for k in range(KMAX):
        chunk_id = wid + k * NW

        @pl.when(chunk_id < NCHUNK)
        def _():
            chunk_work(chunk_id)

    pltpu.sync_copy(acc_v, shared.at[sid])
    plsc.subcore_barrier()

    @pl.when(sid == 0)
    def _():
        lax.fori_loop(0, ACCP // 16, zero_body, 0)

        def red_body(t, _):
            pltpu.sync_copy(shared.at[t], tmp_v)

            def add_body(i, _):
                sl = pl.ds(pl.multiple_of(i * 16, 16), 16)
                acc_v[sl] = acc_v[sl] + tmp_v[sl]
                return 0

            lax.fori_loop(0, ACCP // 16, add_body, 0)
            return 0

        lax.fori_loop(0, 16, red_body, 0)
        pltpu.sync_copy(acc_v, out_hbm.at[cid])


@jax.jit
def _coulomb_sc(charges, d_flat, idx_flat, mol):
    mesh = plsc.VectorSubcoreMesh(core_axis_name="c", subcore_axis_name="s")
    fn = pl.kernel(
        _body,
        out_type=jax.ShapeDtypeStruct((2, ACCP), jnp.float32),
        mesh=mesh,
        compiler_params=pltpu.CompilerParams(needs_layout_passes=False),
        scratch_types=[
            pltpu.VMEM((N,), jnp.float32),
            pltpu.VMEM((R * 32,), jnp.float32),
            pltpu.VMEM((R * 32,), jnp.int32),
            pltpu.VMEM((R,), jnp.int32),
            pltpu.VMEM((ACCP,), jnp.float32),
            pltpu.VMEM((ACCP,), jnp.float32),
            pltpu.VMEM_SHARED((16, ACCP), jnp.float32),
        ],
    )
    return fn(charges, d_flat, idx_flat, mol)


def kernel(charges, d_ij, idx_j, mol_idx):
    charges = charges.astype(jnp.float32)
    d_flat = d_ij.astype(jnp.float32).reshape(-1)
    idx_flat = idx_j.astype(jnp.int32).reshape(-1)
    mol = mol_idx.astype(jnp.int32)
    out = _coulomb_sc(charges, d_flat, idx_flat, mol)
    return (out[0] + out[1])[:NMOL]


# speedup vs baseline: 75.6969x; 1.0540x over previous
import functools

import jax
import jax.numpy as jnp
from jax import lax
from jax.experimental import pallas as pl
from jax.experimental.pallas import tpu as pltpu
from jax.experimental.pallas import tpu_sc as plsc

N = 50000
M = 32
NMOL = 500
RC = 4.6
FACTOR = 13.605693122994 * 0.52917721092

R = 400
NCHUNK = N // R
NW = 32
KMAX = (NCHUNK + NW - 1) // NW
NGROUP = R // 16
ACCP = 512


def _body(chg_hbm, d_hbm, idx_hbm, mol_hbm, out_hbm,
          chg_v, d_v, idx_v, mol_v, acc_v, tmp_v, shared):
    cid = lax.axis_index("c")
    sid = lax.axis_index("s")
    wid = sid * 2 + cid

    pltpu.sync_copy(chg_hbm, chg_v)

    zero16 = jnp.zeros((16,), jnp.float32)

    def zero_body(i, _):
        acc_v[pl.ds(pl.multiple_of(i * 16, 16), 16)] = zero16
        return 0

    lax.fori_loop(0, ACCP // 16, zero_body, 0)

    iota = lax.iota(jnp.int32, 16)
    iota32 = iota * 32
    inv_rc2 = jnp.float32(1.0 / (RC * RC))

    def chunk_work(chunk_id):
        row0 = chunk_id * R
        pltpu.sync_copy(d_hbm.at[pl.ds(pl.multiple_of(row0 * 32, 256), R * 32)], d_v)
        pltpu.sync_copy(idx_hbm.at[pl.ds(pl.multiple_of(row0 * 32, 256), R * 32)], idx_v)
        pltpu.sync_copy(mol_hbm.at[pl.ds(pl.multiple_of(row0, 8), R)], mol_v)

        def group_body(g, _):
            goff = g * (16 * 32)

            s = zero16
            for m in range(M):
                ids = iota32 + (goff + m)
                idxv = plsc.load_gather(idx_v, [ids])
                dv = plsc.load_gather(d_v, [ids])
                qj = plsc.load_gather(chg_v, [idxv])
                x2 = dv * dv * inv_rc2
                inside = x2 < 1.0
                denom = jnp.where(inside, x2 - 1.0, jnp.float32(-1.0))
                fc = jnp.where(inside, 1.0 - jnp.exp(x2 / denom),
                               jnp.float32(1.0))
                s = s + fc * qj / dv

            g16 = g * 16
            qi = chg_v[pl.ds(pl.multiple_of(row0 + g16, 16), 16)]
            e_atom = s * qi * jnp.float32(FACTOR)
            cs = plsc.cumsum(e_atom)
            molv = mol_v[pl.ds(pl.multiple_of(g16, 16), 16)]
            nxt = jnp.minimum(iota + (g16 + 1), R - 1)
            moln = plsc.load_gather(mol_v, [nxt])
            change = molv != moln
            is15 = iota == 15
            endm = change | is15
            boundm = change & jnp.logical_not(is15)
            plsc.addupdate_scatter(acc_v, [molv], cs, mask=endm)
            plsc.addupdate_scatter(acc_v, [moln], -cs, mask=boundm)
            return 0

        lax.fori_loop(0, NGROUP, group_body, 0)

    for k in range(KMAX):
        chunk_id = wid + k * NW

        @pl.when(chunk_id < NCHUNK)
        def _():
            chunk_work(chunk_id)

    pltpu.sync_copy(acc_v, shared.at[sid])
    plsc.subcore_barrier()

    @pl.when(sid == 0)
    def _():
        lax.fori_loop(0, ACCP // 16, zero_body, 0)

        def red_body(t, _):
            pltpu.sync_copy(shared.at[t], tmp_v)

            def add_body(i, _):
                sl = pl.ds(pl.multiple_of(i * 16, 16), 16)
                acc_v[sl] = acc_v[sl] + tmp_v[sl]
                return 0

            lax.fori_loop(0, ACCP // 16, add_body, 0)
            return 0

        lax.fori_loop(0, 16, red_body, 0)
        pltpu.sync_copy(acc_v, out_hbm.at[cid])


@jax.jit
def _coulomb_sc(charges, d_flat, idx_flat, mol):
    mesh = plsc.VectorSubcoreMesh(core_axis_name="c", subcore_axis_name="s")
    fn = pl.kernel(
        _body,
        out_type=jax.ShapeDtypeStruct((2, ACCP), jnp.float32),
        mesh=mesh,
        compiler_params=pltpu.CompilerParams(needs_layout_passes=False),
        scratch_types=[
            pltpu.VMEM((N,), jnp.float32),
            pltpu.VMEM((R * 32,), jnp.float32),
            pltpu.VMEM((R * 32,), jnp.int32),
            pltpu.VMEM((R,), jnp.int32),
            pltpu.VMEM((ACCP,), jnp.float32),
            pltpu.VMEM((ACCP,), jnp.float32),
            pltpu.VMEM_SHARED((16, ACCP), jnp.float32),
        ],
    )
    return fn(charges, d_flat, idx_flat, mol)


def kernel(charges, d_ij, idx_j, mol_idx):
    charges = charges.astype(jnp.float32)
    d_flat = d_ij.astype(jnp.float32).reshape(-1)
    idx_flat = idx_j.astype(jnp.int32).reshape(-1)
    mol = mol_idx.astype(jnp.int32)
    out = _coulomb_sc(charges, d_flat, idx_flat, mol)
    return (out[0] + out[1])[:NMOL]


# speedup vs baseline: 75.7209x; 1.0003x over previous
import functools

import jax
import jax.numpy as jnp
from jax import lax
from jax.experimental import pallas as pl
from jax.experimental.pallas import tpu as pltpu
from jax.experimental.pallas import tpu_sc as plsc

N = 50000
M = 32
NMOL = 500
RC = 4.6
FACTOR = 13.605693122994 * 0.52917721092

R = 400
NCHUNK = N // R
NW = 32
KMAX = (NCHUNK + NW - 1) // NW
NGROUP = R // 16
ACCP = 512


def _body(chg_hbm, d_hbm, idx_hbm, mol_hbm, out_hbm,
          chg_v, d_v, idx_v, mol_v, acc_v, tmp_v, shared):
    cid = lax.axis_index("c")
    sid = lax.axis_index("s")
    wid = sid * 2 + cid

    with jax.named_scope("tbl_copy"):
        pltpu.sync_copy(chg_hbm, chg_v)

    zero16 = jnp.zeros((16,), jnp.float32)

    def zero_body(i, _):
        acc_v[pl.ds(pl.multiple_of(i * 16, 16), 16)] = zero16
        return 0

    lax.fori_loop(0, ACCP // 16, zero_body, 0)

    iota = lax.iota(jnp.int32, 16)
    iota32 = iota * 32
    inv_rc2 = jnp.float32(1.0 / (RC * RC))

    def chunk_work(chunk_id):
        row0 = chunk_id * R
        with jax.named_scope("chunk_dma"):
            pltpu.sync_copy(d_hbm.at[pl.ds(pl.multiple_of(row0 * 32, 256), R * 32)], d_v)
            pltpu.sync_copy(idx_hbm.at[pl.ds(pl.multiple_of(row0 * 32, 256), R * 32)], idx_v)
            pltpu.sync_copy(mol_hbm.at[pl.ds(pl.multiple_of(row0, 8), R)], mol_v)

        def group_body(g, _):
            goff = g * (16 * 32)

            s = zero16
            for m in range(M):
                ids = iota32 + (goff + m)
                idxv = plsc.load_gather(idx_v, [ids])
                dv = plsc.load_gather(d_v, [ids])
                qj = plsc.load_gather(chg_v, [idxv])
                x2 = dv * dv * inv_rc2
                inside = x2 < 1.0
                denom = jnp.where(inside, x2 - 1.0, jnp.float32(-1.0))
                fc = jnp.where(inside, 1.0 - jnp.exp(x2 / denom),
                               jnp.float32(1.0))
                s = s + fc * qj / dv

            g16 = g * 16
            qi = chg_v[pl.ds(pl.multiple_of(row0 + g16, 16), 16)]
            e_atom = s * qi * jnp.float32(FACTOR)
            cs = plsc.cumsum(e_atom)
            molv = mol_v[pl.ds(pl.multiple_of(g16, 16), 16)]
            nxt = jnp.minimum(iota + (g16 + 1), R - 1)
            moln = plsc.load_gather(mol_v, [nxt])
            change = molv != moln
            is15 = iota == 15
            endm = change | is15
            boundm = change & jnp.logical_not(is15)
            plsc.addupdate_scatter(acc_v, [molv], cs, mask=endm)
            plsc.addupdate_scatter(acc_v, [moln], -cs, mask=boundm)
            return 0

        with jax.named_scope("groups"):
            lax.fori_loop(0, NGROUP, group_body, 0)

    for k in range(KMAX):
        chunk_id = wid + k * NW

        @pl.when(chunk_id < NCHUNK)
        def _():
            chunk_work(chunk_id)

    with jax.named_scope("publish"):
        pltpu.sync_copy(acc_v, shared.at[sid])
        plsc.subcore_barrier()

    @pl.when(sid == 0)
    def _():
        lax.fori_loop(0, ACCP // 16, zero_body, 0)

        def red_body(t, _):
            pltpu.sync_copy(shared.at[t], tmp_v)

            def add_body(i, _):
                sl = pl.ds(pl.multiple_of(i * 16, 16), 16)
                acc_v[sl] = acc_v[sl] + tmp_v[sl]
                return 0

            lax.fori_loop(0, ACCP // 16, add_body, 0)
            return 0

        lax.fori_loop(0, 16, red_body, 0)
        pltpu.sync_copy(acc_v, out_hbm.at[cid])


@jax.jit
def _coulomb_sc(charges, d_flat, idx_flat, mol):
    mesh = plsc.VectorSubcoreMesh(core_axis_name="c", subcore_axis_name="s")
    fn = pl.kernel(
        _body,
        out_type=jax.ShapeDtypeStruct((2, ACCP), jnp.float32),
        mesh=mesh,
        compiler_params=pltpu.CompilerParams(needs_layout_passes=False),
        scratch_types=[
            pltpu.VMEM((N,), jnp.float32),
            pltpu.VMEM((R * 32,), jnp.float32),
            pltpu.VMEM((R * 32,), jnp.int32),
            pltpu.VMEM((R,), jnp.int32),
            pltpu.VMEM((ACCP,), jnp.float32),
            pltpu.VMEM((ACCP,), jnp.float32),
            pltpu.VMEM_SHARED((16, ACCP), jnp.float32),
        ],
    )
    return fn(charges, d_flat, idx_flat, mol)


def kernel(charges, d_ij, idx_j, mol_idx):
    charges = charges.astype(jnp.float32)
    d_flat = d_ij.astype(jnp.float32).reshape(-1)
    idx_flat = idx_j.astype(jnp.int32).reshape(-1)
    mol = mol_idx.astype(jnp.int32)
    out = _coulomb_sc(charges, d_flat, idx_flat, mol)
    return (out[0] + out[1])[:NMOL]
